# baseline (speedup 1.0000x reference)
import numpy as np
import jax
import jax.numpy as jnp
from jax.experimental import pallas as pl
from jax.experimental.pallas import tpu as pltpu

TOPK = 4
BLKC = 2048
BLKF = 65536
BLKW = 65536
NBPAD = 512
RG = 8
RPS = 4
BIGI = np.int32(2**30)
NEG = np.float32(-np.inf)
TINY = np.float32(np.finfo(np.float32).tiny)


def _select_topk(vals, idx, k=TOPK):
    out_v, out_i = [], []
    for _ in range(k):
        m = jnp.max(vals, axis=1, keepdims=True)
        sel = jnp.min(jnp.where(vals == m, idx, BIGI), axis=1, keepdims=True)
        out_v.append(m)
        out_i.append(sel)
        vals = jnp.where(idx == sel, NEG, vals)
    return jnp.concatenate(out_v, axis=1), jnp.concatenate(out_i, axis=1)


def _threefry_bits(p_u32):
    k0 = jnp.uint32(0)
    k1 = jnp.uint32(42)
    k2 = k0 ^ k1 ^ jnp.uint32(0x1BD11BDA)
    ks = [k0, k1, k2]
    rot0 = (13, 15, 26, 6)
    rot1 = (17, 29, 16, 24)

    def rotl(x, d):
        return (x << jnp.uint32(d)) | (x >> jnp.uint32(32 - d))

    x0 = jnp.zeros_like(p_u32) + k0
    x1 = p_u32 + k1
    for r in range(5):
        for d in (rot0 if r % 2 == 0 else rot1):
            x0 = x0 + x1
            x1 = rotl(x1, d) ^ x0
        x0 = x0 + ks[(r + 1) % 3]
        x1 = x1 + ks[(r + 2) % 3] + jnp.uint32(r + 1)
    return x0 ^ x1


def _gumbel_at(p_i32):
    bits = _threefry_bits(p_i32.astype(jnp.uint32))
    fb = (bits >> jnp.uint32(9)) | jnp.uint32(0x3F800000)
    f = jax.lax.bitcast_convert_type(fb, jnp.float32) - jnp.float32(1.0)
    u = jnp.maximum(TINY, f + TINY)
    return -jnp.log(-jnp.log(u))


def _make_blockmax_kernel(B, N, nblk, nsteps):
    nsub = BLKF // BLKC

    def body(x_ref, bb_ref, bm):
        step = pl.program_id(0)

        @pl.when(step == 0)
        def _init():
            bm[...] = jnp.full((B, NBPAD), NEG, jnp.float32)

        lane = jax.lax.broadcasted_iota(jnp.int32, (B, NBPAD), 1)

        def merge(x):
            for q in range(nsub):
                k = step * nsub + q
                m = jnp.max(
                    x[:, q * BLKC:(q + 1) * BLKC], axis=1, keepdims=True
                )
                bm[...] = jnp.where(
                    (lane == k) & (k < nblk),
                    jnp.broadcast_to(m, (B, NBPAD)),
                    bm[...],
                )

        merge(x_ref[...])

        @pl.when(step == nsteps - 1)
        def _fix_tail_and_select():
            col = (
                jax.lax.broadcasted_iota(jnp.int32, (B, BLKF), 1) + step * BLKF
            )
            merge(jnp.where(col < N, x_ref[...], NEG))
            bids = jax.lax.broadcasted_iota(jnp.int32, (B, NBPAD), 1)
            _, bb = _select_topk(bm[...], bids)
            bb_ref[...] = bb

    return body


def _make_main_kernel(B, N, nblkw):
    nops = RPS * TOPK

    def body(bb_pref, *refs):
        xrefs = refs[:nops]
        out_ref, s_ref, sv, si, srv = refs[nops:]
        ph = pl.program_id(0)
        s = pl.program_id(1)

        @pl.when((ph == 0) & (s < B // RPS))
        def _scan_rows():
            sub = jax.lax.broadcasted_iota(jnp.int32, (RG, BLKC), 0)
            for t in range(RPS):
                r = s * RPS + t
                picked = []
                for j in range(TOPK):
                    xref = xrefs[t * TOPK + j]
                    xm = jnp.where(sub == r % RG, xref[...], NEG)
                    picked.append(jnp.max(xm, axis=0, keepdims=True))
                x4 = jnp.concatenate(picked, axis=0)
                bids = jnp.concatenate(
                    [
                        jnp.full((1, 1), bb_pref[r * TOPK + j], jnp.int32)
                        for j in range(TOPK)
                    ],
                    axis=0,
                )
                cols = (
                    jax.lax.broadcasted_iota(jnp.int32, (TOPK, BLKC), 1)
                    + bids * BLKC
                )
                x4 = jnp.where(cols < N, x4, NEG)
                bv, bi = _select_topk(x4, cols)
                cv = jnp.concatenate(
                    [bv[q:q + 1, :] for q in range(TOPK)], axis=1
                )
                ci = jnp.concatenate(
                    [bi[q:q + 1, :] for q in range(TOPK)], axis=1
                )
                nv, ni = _select_topk(cv, ci)
                rmask = (
                    jax.lax.broadcasted_iota(jnp.int32, (B, TOPK), 0) == r
                )
                sv[...] = jnp.where(
                    rmask, jnp.broadcast_to(nv, (B, TOPK)), sv[...]
                )
                si[...] = jnp.where(
                    rmask, jnp.broadcast_to(ni, (B, TOPK)), si[...]
                )

        @pl.when(ph == 1)
        def _write():
            @pl.when(s == 0)
            def _finalize():
                v = sv[...]
                ix = si[...]
                rv = v / jnp.sum(v, axis=1, keepdims=True)
                srv[...] = rv
                p = ix + jax.lax.broadcasted_iota(jnp.int32, (B, TOPK), 0) * N
                score = jnp.log(rv + jnp.float32(1e-20)) + _gumbel_at(p)
                m = jnp.max(score, axis=1, keepdims=True)
                j2 = jax.lax.broadcasted_iota(jnp.int32, (B, TOPK), 1)
                jsel = jnp.min(
                    jnp.where(score == m, j2, BIGI), axis=1, keepdims=True
                )
                s_ref[...] = jnp.sum(
                    jnp.where(j2 == jsel, ix, 0), axis=1, keepdims=True
                )

            w = jnp.minimum(s, nblkw - 1)
            col = jax.lax.broadcasted_iota(jnp.int32, (B, BLKW), 1)
            acc = jnp.zeros((B, BLKW), jnp.float32)
            for jj in range(TOPK):
                cj = si[:, pl.ds(jj, 1)] - w * BLKW
                vj = srv[:, pl.ds(jj, 1)]
                acc = jnp.where(col == cj, vj, acc)
            out_ref[...] = acc

    return body


def kernel(softmax):
    B, N = softmax.shape
    nblk = (N + BLKC - 1) // BLKC
    nsteps = (N + BLKF - 1) // BLKF
    nblkw = (N + BLKW - 1) // BLKW
    assert nblk <= NBPAD and B % RPS == 0

    bb = pl.pallas_call(
        _make_blockmax_kernel(B, N, nblk, nsteps),
        grid=(nsteps,),
        in_specs=[pl.BlockSpec((B, BLKF), lambda i: (0, i))],
        out_specs=pl.BlockSpec((B, TOPK), lambda i: (0, 0)),
        out_shape=jax.ShapeDtypeStruct((B, TOPK), jnp.int32),
        scratch_shapes=[pltpu.VMEM((B, NBPAD), jnp.float32)],
    )(softmax)

    G = max(B // RPS, nblkw)

    def cand_spec(t, j):
        return pl.BlockSpec(
            (RG, BLKC),
            lambda ph, s, bb, t=t, j=j: (
                (jnp.minimum(s * RPS + t, B - 1) // RG) * (1 - ph),
                bb[jnp.minimum((s * RPS + t) * TOPK + j, B * TOPK - 1)]
                * (1 - ph),
            ),
        )

    renorm, s2d = pl.pallas_call(
        _make_main_kernel(B, N, nblkw),
        grid_spec=pltpu.PrefetchScalarGridSpec(
            num_scalar_prefetch=1,
            grid=(2, G),
            in_specs=[
                cand_spec(t, j) for t in range(RPS) for j in range(TOPK)
            ],
            out_specs=[
                pl.BlockSpec(
                    (B, BLKW),
                    lambda ph, s, bb: (0, jnp.minimum(s, nblkw - 1) * ph),
                ),
                pl.BlockSpec((B, 1), lambda ph, s, bb: (0, 0)),
            ],
            scratch_shapes=[
                pltpu.VMEM((B, TOPK), jnp.float32),
                pltpu.VMEM((B, TOPK), jnp.int32),
                pltpu.VMEM((B, TOPK), jnp.float32),
            ],
        ),
        out_shape=[
            jax.ShapeDtypeStruct((B, N), jnp.float32),
            jax.ShapeDtypeStruct((B, 1), jnp.int32),
        ],
    )(bb.reshape(-1), *([softmax] * (RPS * TOPK)))

    return renorm, s2d.reshape(B)
